# baseline (speedup 1.0000x reference)
import functools

import jax
import jax.numpy as jnp
import numpy as np
from jax import lax
from jax.experimental import pallas as pl
from jax.experimental.pallas import tpu as pltpu
from jax.experimental.pallas import tpu_sc as plsc

N = 100000
D = 512
NC = 2
NS = 16
NW = NC * NS
CH = 120
NFULL = N // CH
TAIL = N - NFULL * CH
NCHUNKS = NFULL + 1
NPAD = NCHUNKS * CH
WCH = 27

_IDX_CACHE = None


def _perm_idx():
    global _IDX_CACHE
    if _IDX_CACHE is None:
        with jax.ensure_compile_time_eval():
            idx = jax.random.permutation(jax.random.key(42), N)
        perm = np.asarray(jax.device_get(idx), dtype=np.int32)
        _IDX_CACHE = np.concatenate(
            [perm, np.zeros(NPAD - N, dtype=np.int32)])
    return _IDX_CACHE


_mesh = plsc.VectorSubcoreMesh(core_axis_name="c", subcore_axis_name="s")


@functools.partial(
    pl.kernel,
    mesh=_mesh,
    out_type=jax.ShapeDtypeStruct((N, D), jnp.float32),
    scratch_types=[
        pltpu.VMEM((WCH * CH,), jnp.int32),
        pltpu.VMEM((2, CH, D), jnp.float32),
        pltpu.SemaphoreType.DMA,
        pltpu.SemaphoreType.DMA,
    ],
)
def _gather_rows(table_hbm, idx_hbm, out_hbm, idx_v, rows_v, sem0, sem1):
    wid = lax.axis_index("s") * NC + lax.axis_index("c")
    first = WCH * wid - jnp.minimum(wid, 30)
    pltpu.sync_copy(idx_hbm.at[pl.ds(first * CH, WCH * CH)], idx_v)
    base = first * CH
    sems = (sem0, sem1)

    def gather(k, slot):
        pltpu.async_copy(
            table_hbm.at[idx_v.at[pl.ds(k * CH, CH)]],
            rows_v.at[slot], sems[slot])

    def wait_gather(slot):
        pltpu.make_async_copy(
            table_hbm.at[pl.ds(0, CH)], rows_v.at[slot], sems[slot]).wait()

    gather(0, 0)

    def body(t, carry):
        k0 = 2 * t
        gather(k0 + 1, 1)
        wait_gather(0)
        pltpu.sync_copy(rows_v.at[0], out_hbm.at[pl.ds(base + k0 * CH, CH)])
        gather(k0 + 2, 0)
        wait_gather(1)
        pltpu.sync_copy(rows_v.at[1],
                        out_hbm.at[pl.ds(base + (k0 + 1) * CH, CH)])
        return carry

    lax.fori_loop(0, WCH // 2, body, 0)

    wait_gather(0)
    is_last_worker = wid == NW - 1

    @pl.when(is_last_worker)
    def _store_tail():
        pltpu.sync_copy(rows_v.at[0].at[pl.ds(0, TAIL)],
                        out_hbm.at[pl.ds(NFULL * CH, TAIL)])

    @pl.when(jnp.logical_not(is_last_worker))
    def _store_full():
        pltpu.sync_copy(rows_v.at[0],
                        out_hbm.at[pl.ds(base + (WCH - 1) * CH, CH)])


def kernel(features):
    return _gather_rows(features, jnp.asarray(_perm_idx()))


# speedup vs baseline: 1.0631x; 1.0631x over previous
import functools

import jax
import jax.numpy as jnp
import numpy as np
from jax import lax
from jax.experimental import pallas as pl
from jax.experimental.pallas import tpu as pltpu
from jax.experimental.pallas import tpu_sc as plsc

N = 100000
D = 512
NC = 2
NS = 16
NW = NC * NS
CH = 112
NCHUNKS = 893
TAIL = N - 892 * CH
NPAD = NCHUNKS * CH
MAXCH = 28

_IDX_CACHE = None


def _perm_idx():
    global _IDX_CACHE
    if _IDX_CACHE is None:
        with jax.ensure_compile_time_eval():
            idx = jax.random.permutation(jax.random.key(42), N)
        perm = np.asarray(jax.device_get(idx), dtype=np.int32)
        _IDX_CACHE = np.concatenate(
            [perm, np.zeros(NPAD - N, dtype=np.int32)])
    return _IDX_CACHE


_mesh = plsc.VectorSubcoreMesh(core_axis_name="c", subcore_axis_name="s")


@functools.partial(
    pl.kernel,
    mesh=_mesh,
    out_type=jax.ShapeDtypeStruct((N, D), jnp.float32),
    scratch_types=[
        pltpu.VMEM((MAXCH * CH,), jnp.int32),
        pltpu.VMEM((2, CH, D), jnp.float32),
        pltpu.SemaphoreType.DMA,
        pltpu.SemaphoreType.DMA,
    ],
)
def _gather_rows(table_hbm, idx_hbm, out_hbm, idx_v, rows_v, sem0, sem1):
    wid = lax.axis_index("s") * NC + lax.axis_index("c")
    first = MAXCH * wid - jnp.maximum(0, wid - 28)
    base = first * CH
    is_last_worker = wid == NW - 1
    sems = (sem0, sem1)

    def gather(k, slot):
        pltpu.async_copy(
            table_hbm.at[idx_v.at[pl.ds(k * CH, CH)]],
            rows_v.at[slot], sems[slot])

    def wait_gather(slot):
        pltpu.make_async_copy(
            table_hbm.at[pl.ds(0, CH)], rows_v.at[slot], sems[slot]).wait()

    pltpu.sync_copy(idx_hbm.at[pl.ds(base, CH)], idx_v.at[pl.ds(0, CH)])
    gather(0, 0)
    pltpu.sync_copy(idx_hbm.at[pl.ds(base + CH, (MAXCH - 1) * CH)],
                    idx_v.at[pl.ds(CH, (MAXCH - 1) * CH)])

    def body(t, carry):
        k0 = 2 * t
        gather(k0 + 1, 1)
        wait_gather(0)
        pltpu.sync_copy(rows_v.at[0], out_hbm.at[pl.ds(base + k0 * CH, CH)])

        @pl.when(t < MAXCH // 2 - 1)
        def _prefetch():
            gather(k0 + 2, 0)

        wait_gather(1)
        is_tail = is_last_worker & (t == MAXCH // 2 - 1)

        @pl.when(is_tail)
        def _store_tail():
            pltpu.sync_copy(rows_v.at[1].at[pl.ds(0, TAIL)],
                            out_hbm.at[pl.ds(892 * CH, TAIL)])

        @pl.when(jnp.logical_not(is_tail))
        def _store_full():
            pltpu.sync_copy(rows_v.at[1],
                            out_hbm.at[pl.ds(base + (k0 + 1) * CH, CH)])

        return carry

    lax.fori_loop(0, MAXCH // 2, body, 0)


def kernel(features):
    return _gather_rows(features, jnp.asarray(_perm_idx()))
